# baseline (speedup 1.0000x reference)
import jax
import jax.numpy as jnp
from jax.experimental import pallas as pl
from jax.experimental.pallas import tpu as pltpu

_NCHUNK = 8


def _scan_kernel(x_ref, b_ref, g1_ref, g2_ref, g3_ref, o_ref,
                 bxs, g1s, g2s, g3s, os):
    bcb, H, W = x_ref.shape
    wc = W // _NCHUNK
    zero = jnp.zeros((bcb, 1), jnp.float32)

    def relayout_in(c):
        sl = slice(c * wc, (c + 1) * wc)
        bxs[sl] = jnp.transpose(x_ref[:, :, sl] * b_ref[:, :, sl], (2, 0, 1))
        g1s[sl] = jnp.transpose(g1_ref[:, :, sl], (2, 0, 1))
        g2s[sl] = jnp.transpose(g2_ref[:, :, sl], (2, 0, 1))
        g3s[sl] = jnp.transpose(g3_ref[:, :, sl], (2, 0, 1))

    def scan_chunk(c, h):
        for w in range(c * wc, (c + 1) * wc):
            up = jnp.concatenate([zero, h[:, :-1]], axis=1)
            dn = jnp.concatenate([h[:, 1:], zero], axis=1)
            h = bxs[w] + g1s[w] * up + g2s[w] * h + g3s[w] * dn
            os[w] = h
        return h

    def relayout_out(c):
        sl = slice(c * wc, (c + 1) * wc)
        o_ref[:, :, sl] = jnp.transpose(os[sl], (1, 2, 0))

    relayout_in(0)
    if _NCHUNK > 1:
        relayout_in(1)
    h = jnp.zeros((bcb, H), jnp.float32)
    for c in range(_NCHUNK):
        h = scan_chunk(c, h)
        if c + 2 < _NCHUNK:
            relayout_in(c + 2)
        if c >= 1:
            relayout_out(c - 1)
    relayout_out(_NCHUNK - 1)


def kernel(X, B, G1, G2, G3):
    Bsz, C, H, W = X.shape
    BC = Bsz * C
    bcb = min(32, BC)

    ins = [t.reshape(BC, H, W) for t in (X, B, G1, G2, G3)]

    spec = pl.BlockSpec((bcb, H, W), lambda i: (i, 0, 0))
    scratch = [pltpu.VMEM((W, bcb, H), jnp.float32) for _ in range(5)]
    out = pl.pallas_call(
        _scan_kernel,
        grid=(BC // bcb,),
        in_specs=[spec] * 5,
        out_specs=spec,
        out_shape=jax.ShapeDtypeStruct((BC, H, W), jnp.float32),
        scratch_shapes=scratch,
        compiler_params=pltpu.CompilerParams(
            dimension_semantics=("parallel",),
            vmem_limit_bytes=100 * 1024 * 1024,
        ),
    )(*ins)
    return out.reshape(Bsz, C, H, W)


# speedup vs baseline: 3.0182x; 3.0182x over previous
import jax
import jax.numpy as jnp
from jax.experimental import pallas as pl
from jax.experimental.pallas import tpu as pltpu


def _scan_kernel(x_ref, b_ref, g1_ref, g2_ref, g3_ref, o_ref,
                 bxs, g1s, g2s, g3s, os2):
    bcb, H, W = x_ref.shape
    j = pl.program_id(1)
    s = j % 2
    sp = 1 - s

    bxs[s] = jnp.transpose(x_ref[...] * b_ref[...], (2, 0, 1))
    g1s[s] = jnp.transpose(g1_ref[...], (2, 0, 1))
    g2s[s] = jnp.transpose(g2_ref[...], (2, 0, 1))
    g3s[s] = jnp.transpose(g3_ref[...], (2, 0, 1))

    zero = jnp.zeros((bcb, 1), jnp.float32)
    h = jnp.zeros((bcb, H), jnp.float32)
    for w in range(W):
        up = jnp.concatenate([zero, h[:, :-1]], axis=1)
        dn = jnp.concatenate([h[:, 1:], zero], axis=1)
        h = bxs[sp, w] + g1s[sp, w] * up + g2s[sp, w] * h + g3s[sp, w] * dn
        os2[sp, w] = h

    o_ref[...] = jnp.transpose(os2[s], (1, 2, 0))


def kernel(X, B, G1, G2, G3):
    Bsz, C, H, W = X.shape
    BC = Bsz * C
    bcb = min(32, BC)
    NB = BC // bcb
    NC = 2 if NB % 2 == 0 else 1
    P = NB // NC

    ins = [t.reshape(BC, H, W) for t in (X, B, G1, G2, G3)]

    in_spec = pl.BlockSpec((bcb, H, W),
                           lambda c, j: (c * P + jnp.minimum(j, P - 1), 0, 0))
    out_spec = pl.BlockSpec((bcb, H, W),
                            lambda c, j: (c * P + jnp.maximum(j - 2, 0), 0, 0))
    scratch = [pltpu.VMEM((2, W, bcb, H), jnp.float32) for _ in range(5)]
    out = pl.pallas_call(
        _scan_kernel,
        grid=(NC, P + 2),
        in_specs=[in_spec] * 5,
        out_specs=out_spec,
        out_shape=jax.ShapeDtypeStruct((BC, H, W), jnp.float32),
        scratch_shapes=scratch,
        compiler_params=pltpu.CompilerParams(
            dimension_semantics=("parallel", "arbitrary"),
            vmem_limit_bytes=100 * 1024 * 1024,
        ),
    )(*ins)
    return out.reshape(Bsz, C, H, W)


# speedup vs baseline: 3.8813x; 1.2860x over previous
import jax
import jax.numpy as jnp
from jax.experimental import pallas as pl
from jax.experimental.pallas import tpu as pltpu


def _scan_kernel(x_ref, b_ref, g1_ref, g2_ref, g3_ref, o_ref,
                 bxs, g1s, g2s, g3s, os):
    bcb, H, W = x_ref.shape

    bxs[...] = jnp.transpose(x_ref[...] * b_ref[...], (2, 0, 1))
    g1s[...] = jnp.transpose(g1_ref[...], (2, 0, 1))
    g2s[...] = jnp.transpose(g2_ref[...], (2, 0, 1))
    g3s[...] = jnp.transpose(g3_ref[...], (2, 0, 1))

    zero = jnp.zeros((bcb, 1), jnp.float32)
    h = jnp.zeros((bcb, H), jnp.float32)
    for w in range(W):
        up = jnp.concatenate([zero, h[:, :-1]], axis=1)
        dn = jnp.concatenate([h[:, 1:], zero], axis=1)
        h = bxs[w] + g1s[w] * up + g2s[w] * h + g3s[w] * dn
        os[w] = h

    o_ref[...] = jnp.transpose(os[...], (1, 2, 0))


def kernel(X, B, G1, G2, G3):
    Bsz, C, H, W = X.shape
    BC = Bsz * C
    bcb = min(32, BC)

    ins = [t.reshape(BC, H, W) for t in (X, B, G1, G2, G3)]

    spec = pl.BlockSpec((bcb, H, W), lambda i: (i, 0, 0))
    scratch = [pltpu.VMEM((W, bcb, H), jnp.float32) for _ in range(5)]
    out = pl.pallas_call(
        _scan_kernel,
        grid=(BC // bcb,),
        in_specs=[spec] * 5,
        out_specs=spec,
        out_shape=jax.ShapeDtypeStruct((BC, H, W), jnp.float32),
        scratch_shapes=scratch,
        compiler_params=pltpu.CompilerParams(
            dimension_semantics=("parallel",),
            vmem_limit_bytes=100 * 1024 * 1024,
        ),
    )(*ins)
    return out.reshape(Bsz, C, H, W)
